# Initial kernel scaffold; baseline (speedup 1.0000x reference)
import jax
import jax.numpy as jnp
from jax.experimental import pallas as pl


def kernel(x, edge_index, W1, b1, g1, be1, W2, b2, g2, be2):
    raise NotImplementedError("write your pallas kernel here")


# speedup vs baseline: 14.8910x; 14.8910x over previous
import functools

import jax
import jax.numpy as jnp
from jax import lax
from jax.experimental import pallas as pl
from jax.experimental.pallas import tpu as pltpu
from jax.experimental.pallas import tpu_sc as plsc

N = 10000
E = 320000
D = 128

NC = 2
NS = 16
NW = NC * NS
EPW = E // NW
C = 80
NCHUNK = EPW // C
NP = 10240
RPT = NP // NS

R = 2000
GRID = N // R


def _sc_mesh():
    return plsc.VectorSubcoreMesh(
        core_axis_name="c", subcore_axis_name="s",
        num_cores=NC, num_subcores=NS)


DW = 16


def _deg_body(dst_hbm, ones_hbm, zeros_hbm, out_hbm, acc_s, dst_v, ones_v):
    cid = lax.axis_index("c")
    sid = lax.axis_index("s")
    wid = sid * NC + cid
    pltpu.sync_copy(zeros_hbm.at[pl.ds(sid * RPT, RPT)],
                    acc_s.at[pl.ds(sid * RPT, RPT)])
    pltpu.sync_copy(dst_hbm.at[wid], dst_v)
    pltpu.sync_copy(ones_hbm, ones_v)
    plsc.subcore_barrier()

    def body(j, carry):
        pltpu.sync_copy(ones_v, acc_s.at[dst_v.at[j]], add=True)
        return carry

    lax.fori_loop(0, NCHUNK, body, 0)
    plsc.subcore_barrier()
    pltpu.sync_copy(acc_s.at[pl.ds(sid * RPT, RPT)],
                    out_hbm.at[cid, pl.ds(sid * RPT, RPT)])


def _deg_partials(dst_r, ones_c, zeros_ndw):
    k = pl.kernel(
        _deg_body,
        out_type=jax.ShapeDtypeStruct((NC, NP, DW), jnp.float32),
        mesh=_sc_mesh(),
        scratch_types=[
            pltpu.VMEM_SHARED((NP, DW), jnp.float32),
            pltpu.VMEM((NCHUNK, C), jnp.int32),
            pltpu.VMEM((C, DW), jnp.float32),
        ],
    )
    return k(dst_r, ones_c, zeros_ndw)


def _agg_body(hs_hbm, src_hbm, dst_hbm, zeros_hbm, out_hbm,
              acc_s, src_v, dst_v, rows_v, sem):
    cid = lax.axis_index("c")
    sid = lax.axis_index("s")
    wid = sid * NC + cid
    pltpu.sync_copy(zeros_hbm.at[pl.ds(sid * RPT, RPT)],
                    acc_s.at[pl.ds(sid * RPT, RPT)])
    pltpu.sync_copy(src_hbm.at[wid], src_v)
    pltpu.sync_copy(dst_hbm.at[wid], dst_v)
    plsc.subcore_barrier()

    def body(j, carry):
        pltpu.async_copy(hs_hbm.at[src_v.at[j]], rows_v, sem).wait()
        pltpu.sync_copy(rows_v, acc_s.at[dst_v.at[j]], add=True)
        return carry

    lax.fori_loop(0, NCHUNK, body, 0)
    plsc.subcore_barrier()
    pltpu.sync_copy(acc_s.at[pl.ds(sid * RPT, RPT)],
                    out_hbm.at[cid, pl.ds(sid * RPT, RPT)])


def _agg_partials(hs, src_r, dst_r, zeros_nd):
    k = pl.kernel(
        _agg_body,
        out_type=jax.ShapeDtypeStruct((NC, NP, D), jnp.float32),
        mesh=_sc_mesh(),
        scratch_types=[
            pltpu.VMEM_SHARED((NP, D), jnp.float32),
            pltpu.VMEM((NCHUNK, C), jnp.int32),
            pltpu.VMEM((NCHUNK, C), jnp.int32),
            pltpu.VMEM((C, D), jnp.float32),
            pltpu.SemaphoreType.DMA,
        ],
    )
    return k(hs, src_r, dst_r, zeros_nd)


def _isd_body(degp_ref, isd_ref):
    deg = degp_ref[0, :N, 0] + degp_ref[1, :N, 0] + 1.0
    isd_ref[...] = lax.rsqrt(deg)[:, None]


def _tc_isd(degp):
    return pl.pallas_call(
        _isd_body,
        out_shape=jax.ShapeDtypeStruct((N, 1), jnp.float32),
    )(degp)


def _pre_body(x_ref, w_ref, b_ref, isd_ref, hs_ref):
    h = jnp.dot(x_ref[...], w_ref[...],
                preferred_element_type=jnp.float32) + b_ref[...]
    hs_ref[...] = h * isd_ref[...]


def _tc_pre(x, w1, b1, isd):
    return pl.pallas_call(
        _pre_body,
        grid=(GRID,),
        in_specs=[
            pl.BlockSpec((R, D), lambda i: (i, 0)),
            pl.BlockSpec((D, D), lambda i: (0, 0)),
            pl.BlockSpec((1, D), lambda i: (0, 0)),
            pl.BlockSpec((R, 1), lambda i: (i, 0)),
        ],
        out_specs=pl.BlockSpec((R, D), lambda i: (i, 0)),
        out_shape=jax.ShapeDtypeStruct((N, D), jnp.float32),
    )(x, w1, b1.reshape(1, D), isd)


def _comb_body(aggp_ref, hs_ref, isd_ref, agg_ref, sum_ref, sq_ref):
    i = pl.program_id(0)
    a = (aggp_ref[0] + aggp_ref[1] + hs_ref[...]) * isd_ref[...]
    agg_ref[...] = a

    @pl.when(i == 0)
    def _():
        sum_ref[...] = jnp.zeros_like(sum_ref)
        sq_ref[...] = jnp.zeros_like(sq_ref)

    sum_ref[...] += jnp.sum(a, axis=0, keepdims=True)
    sq_ref[...] += jnp.sum(a * a, axis=0, keepdims=True)


def _tc_combine(aggp, hs, isd):
    return pl.pallas_call(
        _comb_body,
        grid=(GRID,),
        in_specs=[
            pl.BlockSpec((NC, R, D), lambda i: (0, i, 0)),
            pl.BlockSpec((R, D), lambda i: (i, 0)),
            pl.BlockSpec((R, 1), lambda i: (i, 0)),
        ],
        out_specs=[
            pl.BlockSpec((R, D), lambda i: (i, 0)),
            pl.BlockSpec((1, D), lambda i: (0, 0)),
            pl.BlockSpec((1, D), lambda i: (0, 0)),
        ],
        out_shape=[
            jax.ShapeDtypeStruct((N, D), jnp.float32),
            jax.ShapeDtypeStruct((1, D), jnp.float32),
            jax.ShapeDtypeStruct((1, D), jnp.float32),
        ],
    )(aggp, hs, isd)


def _bn_mm_body(agg_ref, sum_ref, sq_ref, g_ref, be_ref, w_ref, b_ref,
                isd_ref, h_ref, hsn_ref):
    mean = sum_ref[...] * (1.0 / N)
    var = sq_ref[...] * (1.0 / N) - mean * mean
    inv = lax.rsqrt(var + 1e-5)
    h = jnp.maximum((agg_ref[...] - mean) * inv * g_ref[...] + be_ref[...],
                    0.0)
    h_ref[...] = h
    hsn_ref[...] = (jnp.dot(h, w_ref[...],
                            preferred_element_type=jnp.float32)
                    + b_ref[...]) * isd_ref[...]


def _tc_bn_mm(agg, s0, s1, g, be, w2, b2, isd):
    return pl.pallas_call(
        _bn_mm_body,
        grid=(GRID,),
        in_specs=[
            pl.BlockSpec((R, D), lambda i: (i, 0)),
            pl.BlockSpec((1, D), lambda i: (0, 0)),
            pl.BlockSpec((1, D), lambda i: (0, 0)),
            pl.BlockSpec((1, D), lambda i: (0, 0)),
            pl.BlockSpec((1, D), lambda i: (0, 0)),
            pl.BlockSpec((D, D), lambda i: (0, 0)),
            pl.BlockSpec((1, D), lambda i: (0, 0)),
            pl.BlockSpec((R, 1), lambda i: (i, 0)),
        ],
        out_specs=[
            pl.BlockSpec((R, D), lambda i: (i, 0)),
            pl.BlockSpec((R, D), lambda i: (i, 0)),
        ],
        out_shape=[
            jax.ShapeDtypeStruct((N, D), jnp.float32),
            jax.ShapeDtypeStruct((N, D), jnp.float32),
        ],
    )(agg, s0, s1, g.reshape(1, D), be.reshape(1, D), w2,
      b2.reshape(1, D), isd)


def _bn_res_body(agg_ref, sum_ref, sq_ref, g_ref, be_ref, h1_ref, out_ref):
    mean = sum_ref[...] * (1.0 / N)
    var = sq_ref[...] * (1.0 / N) - mean * mean
    inv = lax.rsqrt(var + 1e-5)
    h2 = jnp.maximum((agg_ref[...] - mean) * inv * g_ref[...] + be_ref[...],
                     0.0)
    out_ref[...] = h2 + h1_ref[...]


def _tc_bn_res(agg, s0, s1, g, be, h1):
    return pl.pallas_call(
        _bn_res_body,
        grid=(GRID,),
        in_specs=[
            pl.BlockSpec((R, D), lambda i: (i, 0)),
            pl.BlockSpec((1, D), lambda i: (0, 0)),
            pl.BlockSpec((1, D), lambda i: (0, 0)),
            pl.BlockSpec((1, D), lambda i: (0, 0)),
            pl.BlockSpec((1, D), lambda i: (0, 0)),
            pl.BlockSpec((R, D), lambda i: (i, 0)),
        ],
        out_specs=pl.BlockSpec((R, D), lambda i: (i, 0)),
        out_shape=jax.ShapeDtypeStruct((N, D), jnp.float32),
    )(agg, s0, s1, g.reshape(1, D), be.reshape(1, D), h1)


def kernel(x, edge_index, W1, b1, g1, be1, W2, b2, g2, be2):
    src_r = edge_index[0].reshape(NW, NCHUNK, C)
    dst_r = edge_index[1].reshape(NW, NCHUNK, C)
    ones_c = jnp.ones((C, DW), jnp.float32)
    zeros_ndw = jnp.zeros((NP, DW), jnp.float32)
    zeros_nd = jnp.zeros((NP, D), jnp.float32)

    degp = _deg_partials(dst_r, ones_c, zeros_ndw)
    isd = _tc_isd(degp)
    hs1 = _tc_pre(x, W1, b1, isd)
    aggp1 = _agg_partials(hs1, src_r, dst_r, zeros_nd)
    agg1, s10, s11 = _tc_combine(aggp1, hs1, isd)
    h1, hs2 = _tc_bn_mm(agg1, s10, s11, g1, be1, W2, b2, isd)
    aggp2 = _agg_partials(hs2, src_r, dst_r, zeros_nd)
    agg2, s20, s21 = _tc_combine(aggp2, hs2, isd)
    return _tc_bn_res(agg2, s20, s21, g2, be2, h1)


# speedup vs baseline: 22.3977x; 1.5041x over previous
import functools

import jax
import jax.numpy as jnp
from jax import lax
from jax.experimental import pallas as pl
from jax.experimental.pallas import tpu as pltpu
from jax.experimental.pallas import tpu_sc as plsc

N = 10000
E = 320000
D = 128

NC = 2
NS = 16
NW = NC * NS
EPW = E // NW
C = 80
NCHUNK = EPW // C
NP = 10240
RPT = NP // NS

R = 2000
GRID = N // R


def _sc_mesh():
    return plsc.VectorSubcoreMesh(
        core_axis_name="c", subcore_axis_name="s",
        num_cores=NC, num_subcores=NS)


DW = 16


DK = 8


def _deg_body(dst_hbm, ones_hbm, zeros_hbm, out_hbm, acc_s, dst_v, ones_v,
              *sems):
    cid = lax.axis_index("c")
    sid = lax.axis_index("s")
    wid = sid * NC + cid
    pltpu.sync_copy(zeros_hbm.at[pl.ds(sid * RPT, RPT)],
                    acc_s.at[pl.ds(sid * RPT, RPT)])
    pltpu.sync_copy(dst_hbm.at[wid], dst_v)
    pltpu.sync_copy(ones_hbm, ones_v)
    plsc.subcore_barrier()

    def drain(jd, b):
        pltpu.make_async_copy(ones_v, acc_s.at[dst_v.at[jd]],
                              sems[b]).wait()

    def step(j, b):
        @pl.when(j >= DK)
        def _():
            drain(j - DK, b)
        pltpu.async_copy(ones_v, acc_s.at[dst_v.at[j]], sems[b], add=True)

    def body(j0, carry):
        j = j0 * DK
        for b in range(DK):
            step(j + b, b)
        return carry

    nfull = (NCHUNK // DK) * DK
    lax.fori_loop(0, NCHUNK // DK, body, 0)
    for jt in range(nfull, NCHUNK):
        step(jt, jt % DK)
    for b in range(DK):
        last = max(c for c in range(NCHUNK) if c % DK == b)
        drain(last, b)
    plsc.subcore_barrier()
    pltpu.sync_copy(acc_s.at[pl.ds(sid * RPT, RPT)],
                    out_hbm.at[cid, pl.ds(sid * RPT, RPT)])


def _deg_partials(dst_r, ones_c, zeros_ndw):
    k = pl.kernel(
        _deg_body,
        out_type=jax.ShapeDtypeStruct((NC, NP, DW), jnp.float32),
        mesh=_sc_mesh(),
        scratch_types=[
            pltpu.VMEM_SHARED((NP, DW), jnp.float32),
            pltpu.VMEM((NCHUNK, C), jnp.int32),
            pltpu.VMEM((C, DW), jnp.float32),
        ] + [pltpu.SemaphoreType.DMA] * DK,
    )
    return k(dst_r, ones_c, zeros_ndw)


RI = 8


def _agg_body(hs_hbm, srcf_hbm, dstf_hbm, zeros_hbm, out_hbm,
              acc_s, srcr_v, dstr_v, rows_v, *sems):
    gs = sems[0:4]
    ss = sems[4:8]
    isem = sems[8:12]
    dsem = sems[12:16]
    cid = lax.axis_index("c")
    sid = lax.axis_index("s")
    wid = sid * NC + cid
    ebase = wid * EPW
    pltpu.sync_copy(zeros_hbm.at[pl.ds(sid * RPT, RPT)],
                    acc_s.at[pl.ds(sid * RPT, RPT)])
    plsc.subcore_barrier()

    def idx_issue(j, b):
        r = j % RI
        pltpu.async_copy(srcf_hbm.at[pl.ds(ebase + j * C, C)],
                         srcr_v.at[r], isem[b])
        pltpu.async_copy(dstf_hbm.at[pl.ds(ebase + j * C, C)],
                         dstr_v.at[r], dsem[b])

    def drain_idx(sem):
        pltpu.make_async_copy(srcf_hbm.at[pl.ds(0, C)], srcr_v.at[0],
                              sem).wait()

    def drain_gather(j, b):
        pltpu.make_async_copy(hs_hbm.at[srcr_v.at[j % RI]], rows_v.at[b],
                              gs[b]).wait()

    def drain_scatter(jd, b):
        pltpu.make_async_copy(rows_v.at[b], acc_s.at[dstr_v.at[jd % RI]],
                              ss[b]).wait()

    def gather(j, b):
        pltpu.async_copy(hs_hbm.at[srcr_v.at[j % RI]], rows_v.at[b], gs[b])

    def step(j, b):
        bg = (b + 2) % 4
        drain_gather(j, b)
        pltpu.async_copy(rows_v.at[b], acc_s.at[dstr_v.at[j % RI]], ss[b],
                         add=True)

        @pl.when(j >= 2)
        def _():
            drain_scatter(j - 2, bg)

        @pl.when(j + 4 < NCHUNK)
        def _():
            idx_issue(j + 4, b)

        @pl.when(j + 2 < NCHUNK)
        def _():
            drain_idx(isem[bg])
            drain_idx(dsem[bg])
            gather(j + 2, bg)

    for c0 in range(4):
        idx_issue(c0, c0)
    for c0 in range(2):
        drain_idx(isem[c0])
        drain_idx(dsem[c0])
        gather(c0, c0)

    def body(j0, carry):
        j = j0 * 4
        for b in range(4):
            step(j + b, b)
        return carry

    nfull = (NCHUNK // 4) * 4
    lax.fori_loop(0, NCHUNK // 4, body, 0)
    for jt in range(nfull, NCHUNK):
        step(jt, jt % 4)
    for jt in range(NCHUNK - 2, NCHUNK):
        drain_scatter(jt, jt % 4)
    plsc.subcore_barrier()
    pltpu.sync_copy(acc_s.at[pl.ds(sid * RPT, RPT)],
                    out_hbm.at[cid, pl.ds(sid * RPT, RPT)])


def _agg_partials(hs, src_flat, dst_flat, zeros_nd):
    k = pl.kernel(
        _agg_body,
        out_type=jax.ShapeDtypeStruct((NC, NP, D), jnp.float32),
        mesh=_sc_mesh(),
        scratch_types=[
            pltpu.VMEM_SHARED((NP, D), jnp.float32),
            pltpu.VMEM((RI, C), jnp.int32),
            pltpu.VMEM((RI, C), jnp.int32),
            pltpu.VMEM((4, C, D), jnp.float32),
        ] + [pltpu.SemaphoreType.DMA] * 16,
    )
    return k(hs, src_flat, dst_flat, zeros_nd)


def _isd_body(degp_ref, isd_ref):
    deg = degp_ref[0, :N, 0] + degp_ref[1, :N, 0] + 1.0
    isd_ref[...] = lax.rsqrt(deg)[:, None]


def _tc_isd(degp):
    return pl.pallas_call(
        _isd_body,
        out_shape=jax.ShapeDtypeStruct((N, 1), jnp.float32),
    )(degp)


def _pre_body(x_ref, w_ref, b_ref, isd_ref, hs_ref):
    h = jnp.dot(x_ref[...], w_ref[...],
                preferred_element_type=jnp.float32) + b_ref[...]
    hs_ref[...] = h * isd_ref[...]


def _tc_pre(x, w1, b1, isd):
    return pl.pallas_call(
        _pre_body,
        grid=(GRID,),
        in_specs=[
            pl.BlockSpec((R, D), lambda i: (i, 0)),
            pl.BlockSpec((D, D), lambda i: (0, 0)),
            pl.BlockSpec((1, D), lambda i: (0, 0)),
            pl.BlockSpec((R, 1), lambda i: (i, 0)),
        ],
        out_specs=pl.BlockSpec((R, D), lambda i: (i, 0)),
        out_shape=jax.ShapeDtypeStruct((N, D), jnp.float32),
    )(x, w1, b1.reshape(1, D), isd)


def _comb_body(aggp_ref, hs_ref, isd_ref, agg_ref, sum_ref, sq_ref):
    i = pl.program_id(0)
    a = (aggp_ref[0] + aggp_ref[1] + hs_ref[...]) * isd_ref[...]
    agg_ref[...] = a

    @pl.when(i == 0)
    def _():
        sum_ref[...] = jnp.zeros_like(sum_ref)
        sq_ref[...] = jnp.zeros_like(sq_ref)

    sum_ref[...] += jnp.sum(a, axis=0, keepdims=True)
    sq_ref[...] += jnp.sum(a * a, axis=0, keepdims=True)


def _tc_combine(aggp, hs, isd):
    return pl.pallas_call(
        _comb_body,
        grid=(GRID,),
        in_specs=[
            pl.BlockSpec((NC, R, D), lambda i: (0, i, 0)),
            pl.BlockSpec((R, D), lambda i: (i, 0)),
            pl.BlockSpec((R, 1), lambda i: (i, 0)),
        ],
        out_specs=[
            pl.BlockSpec((R, D), lambda i: (i, 0)),
            pl.BlockSpec((1, D), lambda i: (0, 0)),
            pl.BlockSpec((1, D), lambda i: (0, 0)),
        ],
        out_shape=[
            jax.ShapeDtypeStruct((N, D), jnp.float32),
            jax.ShapeDtypeStruct((1, D), jnp.float32),
            jax.ShapeDtypeStruct((1, D), jnp.float32),
        ],
    )(aggp, hs, isd)


def _bn_mm_body(agg_ref, sum_ref, sq_ref, g_ref, be_ref, w_ref, b_ref,
                isd_ref, h_ref, hsn_ref):
    mean = sum_ref[...] * (1.0 / N)
    var = sq_ref[...] * (1.0 / N) - mean * mean
    inv = lax.rsqrt(var + 1e-5)
    h = jnp.maximum((agg_ref[...] - mean) * inv * g_ref[...] + be_ref[...],
                    0.0)
    h_ref[...] = h
    hsn_ref[...] = (jnp.dot(h, w_ref[...],
                            preferred_element_type=jnp.float32)
                    + b_ref[...]) * isd_ref[...]


def _tc_bn_mm(agg, s0, s1, g, be, w2, b2, isd):
    return pl.pallas_call(
        _bn_mm_body,
        grid=(GRID,),
        in_specs=[
            pl.BlockSpec((R, D), lambda i: (i, 0)),
            pl.BlockSpec((1, D), lambda i: (0, 0)),
            pl.BlockSpec((1, D), lambda i: (0, 0)),
            pl.BlockSpec((1, D), lambda i: (0, 0)),
            pl.BlockSpec((1, D), lambda i: (0, 0)),
            pl.BlockSpec((D, D), lambda i: (0, 0)),
            pl.BlockSpec((1, D), lambda i: (0, 0)),
            pl.BlockSpec((R, 1), lambda i: (i, 0)),
        ],
        out_specs=[
            pl.BlockSpec((R, D), lambda i: (i, 0)),
            pl.BlockSpec((R, D), lambda i: (i, 0)),
        ],
        out_shape=[
            jax.ShapeDtypeStruct((N, D), jnp.float32),
            jax.ShapeDtypeStruct((N, D), jnp.float32),
        ],
    )(agg, s0, s1, g.reshape(1, D), be.reshape(1, D), w2,
      b2.reshape(1, D), isd)


def _bn_res_body(agg_ref, sum_ref, sq_ref, g_ref, be_ref, h1_ref, out_ref):
    mean = sum_ref[...] * (1.0 / N)
    var = sq_ref[...] * (1.0 / N) - mean * mean
    inv = lax.rsqrt(var + 1e-5)
    h2 = jnp.maximum((agg_ref[...] - mean) * inv * g_ref[...] + be_ref[...],
                     0.0)
    out_ref[...] = h2 + h1_ref[...]


def _tc_bn_res(agg, s0, s1, g, be, h1):
    return pl.pallas_call(
        _bn_res_body,
        grid=(GRID,),
        in_specs=[
            pl.BlockSpec((R, D), lambda i: (i, 0)),
            pl.BlockSpec((1, D), lambda i: (0, 0)),
            pl.BlockSpec((1, D), lambda i: (0, 0)),
            pl.BlockSpec((1, D), lambda i: (0, 0)),
            pl.BlockSpec((1, D), lambda i: (0, 0)),
            pl.BlockSpec((R, D), lambda i: (i, 0)),
        ],
        out_specs=pl.BlockSpec((R, D), lambda i: (i, 0)),
        out_shape=jax.ShapeDtypeStruct((N, D), jnp.float32),
    )(agg, s0, s1, g.reshape(1, D), be.reshape(1, D), h1)


def kernel(x, edge_index, W1, b1, g1, be1, W2, b2, g2, be2):
    dst_r = edge_index[1].reshape(NW, NCHUNK, C)
    src_flat = edge_index[0]
    dst_flat = edge_index[1]
    ones_c = jnp.ones((C, DW), jnp.float32)
    zeros_ndw = jnp.zeros((NP, DW), jnp.float32)
    zeros_nd = jnp.zeros((NP, D), jnp.float32)

    degp = _deg_partials(dst_r, ones_c, zeros_ndw)
    isd = _tc_isd(degp)
    hs1 = _tc_pre(x, W1, b1, isd)
    aggp1 = _agg_partials(hs1, src_flat, dst_flat, zeros_nd)
    agg1, s10, s11 = _tc_combine(aggp1, hs1, isd)
    h1, hs2 = _tc_bn_mm(agg1, s10, s11, g1, be1, W2, b2, isd)
    aggp2 = _agg_partials(hs2, src_flat, dst_flat, zeros_nd)
    agg2, s20, s21 = _tc_combine(aggp2, hs2, isd)
    return _tc_bn_res(agg2, s20, s21, g2, be2, h1)
